# baseline (speedup 1.0000x reference)
import functools

import jax
import jax.numpy as jnp
from jax import lax
from jax.experimental import pallas as pl
from jax.experimental.pallas import tpu as pltpu
from jax.experimental.pallas import tpu_sc as plsc

NUM_EXPERTS = 8
LATENT = 16
TB = 2048
NW = 32
CHUNK = 512
CPB = TB // CHUNK


def _tc_body(x_ref, gw_ref, lat_ref):
    x = x_ref[...]
    gw = gw_ref[...]
    latT = jax.lax.dot_general(
        gw, x, (((1,), (1,)), ((), ())),
        preferred_element_type=jnp.float32)
    for c in range(CPB):
        lat_ref[c] = latT[:, c * CHUNK:(c + 1) * CHUNK]


def _tree16(vals):
    v = list(vals)
    while len(v) > 1:
        v = [v[i] + v[i + 1] for i in range(0, len(v), 2)]
    return v[0]


def _sc_route(lat_hbm, ew_hbm, bi_hbm, bj_hbm, lat_v, ew_v, bi_v, bj_v):
    c = lax.axis_index("c")
    s = lax.axis_index("s")
    wid = s * 2 + c
    pltpu.sync_copy(lat_hbm.at[wid], lat_v)

    def group(g, carry):
        base = g * 16
        le = [lat_v[r, pl.ds(base, 16)] for r in range(128)]
        nlb = []
        for e in range(NUM_EXPERTS):
            rows = le[e * LATENT:(e + 1) * LATENT]
            n2 = _tree16([r_ * r_ for r_ in rows])
            i = lax.bitcast_convert_type(n2, jnp.int32)
            i = jnp.int32(0x5F3759DF) - (i >> 1)
            r = lax.bitcast_convert_type(i, jnp.float32)
            h = jnp.float32(0.5) * n2
            for _ in range(3):
                r = r * (jnp.float32(1.5) - h * r * r)
            y = n2 * r
            d = jnp.maximum(y, jnp.float32(1e-12))
            def _round_bf16(v):
                iv = lax.bitcast_convert_type(v, jnp.int32)
                iv = (iv + jnp.int32(0x7FFF) + ((iv >> 16) & jnp.int32(1)))
                iv = iv & jnp.int32(0xFFFF0000 - (1 << 32))
                return lax.bitcast_convert_type(iv, jnp.float32)
            nlb.append([_round_bf16(rw / d) for rw in rows])
        m = jnp.full((16,), -jnp.inf, dtype=jnp.float32)
        bi = jnp.zeros((16,), dtype=jnp.int32)
        bj = jnp.zeros((16,), dtype=jnp.int32)
        for i_ in range(NUM_EXPERTS):
            for j_ in range(i_ + 1, NUM_EXPERTS):
                sv = _tree16([a * b for a, b in zip(nlb[i_], nlb[j_])])
                take = sv > m
                m = jnp.where(take, sv, m)
                bi = jnp.where(take, jnp.int32(i_), bi)
                bj = jnp.where(take, jnp.int32(j_), bj)
        ew_v[pl.ds(base, 16)] = m
        bi_v[pl.ds(base, 16)] = bi
        bj_v[pl.ds(base, 16)] = bj
        return carry

    lax.fori_loop(0, CHUNK // 16, group, 0)

    pltpu.sync_copy(ew_v, ew_hbm.at[wid])
    pltpu.sync_copy(bi_v, bi_hbm.at[wid])
    pltpu.sync_copy(bj_v, bj_hbm.at[wid])


@jax.jit
def kernel(hidden_states, gate_w):
    b, s, h = hidden_states.shape
    n = b * s
    x = hidden_states.reshape(n, h)

    grid = n // TB
    latc = pl.pallas_call(
        _tc_body,
        grid=(grid,),
        in_specs=[
            pl.BlockSpec((TB, h), lambda i: (i, 0)),
            pl.BlockSpec((NUM_EXPERTS * LATENT, h), lambda i: (0, 0)),
        ],
        out_specs=pl.BlockSpec((CPB, 128, CHUNK), lambda i: (i, 0, 0)),
        out_shape=jax.ShapeDtypeStruct((NW, 128, CHUNK), jnp.float32),
    )(x, gate_w)

    mesh = plsc.VectorSubcoreMesh(core_axis_name="c", subcore_axis_name="s")
    route = functools.partial(
        pl.kernel,
        mesh=mesh,
        out_type=[
            jax.ShapeDtypeStruct((NW, CHUNK), jnp.float32),
            jax.ShapeDtypeStruct((NW, CHUNK), jnp.int32),
            jax.ShapeDtypeStruct((NW, CHUNK), jnp.int32),
        ],
        scratch_types=[
            pltpu.VMEM((128, CHUNK), jnp.float32),
            pltpu.VMEM((CHUNK,), jnp.float32),
            pltpu.VMEM((CHUNK,), jnp.int32),
            pltpu.VMEM((CHUNK,), jnp.int32),
        ],
    )(_sc_route)
    ew, bi, bj = route(latc)

    expert_weights = ew.reshape(n)
    selected_experts = jnp.stack([bi.reshape(n), bj.reshape(n)], axis=1)
    return (expert_weights, selected_experts)


# speedup vs baseline: 1.8255x; 1.8255x over previous
import jax
import jax.numpy as jnp
from jax.experimental import pallas as pl

NUM_EXPERTS = 8
LATENT = 16
TB = 2048


def _body(x_ref, gw_ref, ew_ref, bi_ref, bj_ref):
    x = x_ref[...]
    gw = gw_ref[...]
    latT = jax.lax.dot_general(
        gw, x, (((1,), (1,)), ((), ())),
        preferred_element_type=jnp.float32)

    nl = []
    for e in range(NUM_EXPERTS):
        le = latT[e * LATENT:(e + 1) * LATENT, :]
        n2 = jnp.sum(le * le, axis=0, keepdims=True)
        denom = jnp.maximum(jnp.sqrt(n2), 1e-12)
        nle = le / denom
        nl.append(nle.astype(jnp.bfloat16).astype(jnp.float32))

    m = jnp.full((1, TB), -jnp.inf, dtype=jnp.float32)
    bi = jnp.zeros((1, TB), dtype=jnp.int32)
    bj = jnp.zeros((1, TB), dtype=jnp.int32)
    for i in range(NUM_EXPERTS):
        for j in range(i + 1, NUM_EXPERTS):
            s = jnp.sum(nl[i] * nl[j], axis=0, keepdims=True)
            take = s > m
            m = jnp.where(take, s, m)
            bi = jnp.where(take, jnp.int32(i), bi)
            bj = jnp.where(take, jnp.int32(j), bj)

    ew_ref[0, :, :] = m
    bi_ref[0, :, :] = bi
    bj_ref[0, :, :] = bj


@jax.jit
def kernel(hidden_states, gate_w):
    b, s, h = hidden_states.shape
    n = b * s
    x = hidden_states.reshape(n, h)

    grid = n // TB
    ew, bi, bj = pl.pallas_call(
        _body,
        grid=(grid,),
        in_specs=[
            pl.BlockSpec((TB, h), lambda i: (i, 0)),
            pl.BlockSpec((NUM_EXPERTS * LATENT, h), lambda i: (0, 0)),
        ],
        out_specs=[
            pl.BlockSpec((1, 1, TB), lambda i: (i, 0, 0)),
            pl.BlockSpec((1, 1, TB), lambda i: (i, 0, 0)),
            pl.BlockSpec((1, 1, TB), lambda i: (i, 0, 0)),
        ],
        out_shape=[
            jax.ShapeDtypeStruct((grid, 1, TB), jnp.float32),
            jax.ShapeDtypeStruct((grid, 1, TB), jnp.int32),
            jax.ShapeDtypeStruct((grid, 1, TB), jnp.int32),
        ],
    )(x, gate_w)

    expert_weights = ew.reshape(n)
    selected_experts = jnp.stack([bi.reshape(n), bj.reshape(n)], axis=1)
    return (expert_weights, selected_experts)
